# Initial kernel scaffold; baseline (speedup 1.0000x reference)
import jax
import jax.numpy as jnp
from jax.experimental import pallas as pl


def kernel(x_loc, x_expert, edge_index, W_l, b_l, W_r, W_lin, b_lin):
    raise NotImplementedError("write your pallas kernel here")


# speedup vs baseline: 4.9329x; 4.9329x over previous
import functools

import jax
import jax.numpy as jnp
from jax import lax
from jax.experimental import pallas as pl
from jax.experimental.pallas import tpu as pltpu
from jax.experimental.pallas import tpu_sc as plsc

N_LOC = 10000
N_EXP = 10000
E = 320000
D = 128
H = 128
OUT = 128

NC = 2
NS = 16
NW = NC * NS
K = 128
NCH = -(-E // (NW * K))
EPW = NCH * K
E_PAD = NW * EPW
N_ACC = 10240
RPS = N_ACC // NS
G16 = K // 16

_sc_mesh = plsc.VectorSubcoreMesh(core_axis_name="c", subcore_axis_name="s")


@functools.partial(
    pl.kernel,
    mesh=_sc_mesh,
    compiler_params=pltpu.CompilerParams(needs_layout_passes=False),
    out_type=(
        jax.ShapeDtypeStruct((NC, N_ACC, D), jnp.float32),
        jax.ShapeDtypeStruct((NC, N_ACC), jnp.float32),
    ),
    scratch_types=[
        pltpu.VMEM((K,), jnp.int32),
        pltpu.VMEM((K,), jnp.int32),
        pltpu.VMEM((K, D), jnp.float32),
        pltpu.VMEM((N_ACC,), jnp.float32),
        pltpu.VMEM((NS, RPS), jnp.float32),
        pltpu.VMEM((RPS,), jnp.float32),
        pltpu.VMEM_SHARED((N_ACC, D), jnp.float32),
        pltpu.VMEM_SHARED((NS, N_ACC), jnp.float32),
        pltpu.SemaphoreType.DMA,
    ],
)
def _sc_segment_sum(x_loc, src_p, dst_p, zrows, out_feat, out_cnt,
                    src_v, dst_v, rows_v, hist_v, slab_v, cred_v,
                    acc_sh, cnt_sh, sem):
    c = lax.axis_index("c")
    s = lax.axis_index("s")
    wid = s * NC + c

    zeros16 = jnp.zeros((16,), jnp.float32)

    def zh(k, carry):
        hist_v[pl.ds(k * 16, 16)] = zeros16
        return carry

    lax.fori_loop(0, N_ACC // 16, zh, 0)
    pltpu.sync_copy(zrows.at[pl.ds(s * RPS, RPS)],
                    acc_sh.at[pl.ds(s * RPS, RPS)])
    plsc.subcore_barrier()

    ones16 = jnp.ones((16,), jnp.float32)

    def body(i, carry):
        base = wid * EPW + i * K
        pltpu.sync_copy(src_p.at[pl.ds(base, K)], src_v)
        pltpu.sync_copy(dst_p.at[pl.ds(base, K)], dst_v)
        pltpu.async_copy(x_loc.at[src_v], rows_v, sem).wait()
        pltpu.sync_copy(rows_v, acc_sh.at[dst_v], add=True)
        for g in range(G16):
            idx16 = dst_v[pl.ds(g * 16, 16)]
            plsc.addupdate_scatter(hist_v, [idx16], ones16)
        return carry

    lax.fori_loop(0, NCH, body, 0)

    pltpu.sync_copy(hist_v, cnt_sh.at[s])
    plsc.subcore_barrier()
    pltpu.sync_copy(cnt_sh.at[:, pl.ds(s * RPS, RPS)], slab_v)

    def cr(g, carry):
        acc = slab_v[0, pl.ds(g * 16, 16)]
        for j in range(1, NS):
            acc = acc + slab_v[j, pl.ds(g * 16, 16)]
        cred_v[pl.ds(g * 16, 16)] = acc
        return carry

    lax.fori_loop(0, RPS // 16, cr, 0)

    pltpu.sync_copy(acc_sh.at[pl.ds(s * RPS, RPS)],
                    out_feat.at[c].at[pl.ds(s * RPS, RPS)])
    pltpu.sync_copy(cred_v, out_cnt.at[c].at[pl.ds(s * RPS, RPS)])


def _tc_body(p0, p1, c0, c1, xe, wl, wr, wo, bl, bo, o):
    cnt = jnp.maximum(c0[0] + c1[0], 1.0)
    sacc = p0[0] + p1[0]
    mean = sacc / cnt
    h = jnp.dot(mean, wl[...], preferred_element_type=jnp.float32)
    h = h + jnp.dot(xe[...], wr[...], preferred_element_type=jnp.float32)
    h = jnp.maximum(h + bl[...], 0.0)
    o[...] = jnp.dot(h, wo[...], preferred_element_type=jnp.float32) + bo[...]


BT = 1024


def _tc_stage(parts, cnts, x_expert, wlT, wrT, woT, bl, bo):
    grid = (-(-N_EXP // BT),)
    return pl.pallas_call(
        _tc_body,
        grid=grid,
        in_specs=[
            pl.BlockSpec((1, BT, D), lambda i: (0, i, 0)),
            pl.BlockSpec((1, BT, D), lambda i: (1, i, 0)),
            pl.BlockSpec((1, BT, 1), lambda i: (0, i, 0)),
            pl.BlockSpec((1, BT, 1), lambda i: (1, i, 0)),
            pl.BlockSpec((BT, D), lambda i: (i, 0)),
            pl.BlockSpec((D, H), lambda i: (0, 0)),
            pl.BlockSpec((D, H), lambda i: (0, 0)),
            pl.BlockSpec((H, OUT), lambda i: (0, 0)),
            pl.BlockSpec((1, H), lambda i: (0, 0)),
            pl.BlockSpec((1, OUT), lambda i: (0, 0)),
        ],
        out_specs=pl.BlockSpec((BT, OUT), lambda i: (i, 0)),
        out_shape=jax.ShapeDtypeStruct((N_EXP, OUT), jnp.float32),
    )(parts, parts, cnts, cnts, x_expert, wlT, wrT, woT, bl, bo)


def kernel(x_loc, x_expert, edge_index, W_l, b_l, W_r, W_lin, b_lin):
    src = edge_index[0]
    dst = edge_index[1]
    pad = E_PAD - E
    src_p = jnp.concatenate([src, jnp.zeros((pad,), jnp.int32)])
    dst_p = jnp.concatenate([dst, jnp.full((pad,), N_EXP, jnp.int32)])
    zrows = jnp.zeros((N_ACC, D), jnp.float32)

    parts, cnts = _sc_segment_sum(x_loc, src_p, dst_p, zrows)
    return _tc_stage(parts, cnts.reshape(NC, N_ACC, 1), x_expert,
                     W_l.T, W_r.T, W_lin.T, b_l[None, :], b_lin[None, :])
